# baseline (speedup 1.0000x reference)
import functools

import jax
import jax.numpy as jnp
from jax import lax
from jax.experimental import pallas as pl
from jax.experimental.pallas import tpu as pltpu
from jax.experimental.pallas import tpu_sc as plsc

_NBUF = 4
_L = 16


def _tc_detile_fn(v, d):
    assert d == 64
    grid = (v + 127) // 128
    vw = grid * d

    def body(x_ref, o_ref):
        xt = jnp.swapaxes(x_ref[...], 0, 1)
        o_ref[...] = jnp.concatenate([xt[:d], xt[d:]], axis=1)

    return pl.pallas_call(
        body,
        grid=(grid,),
        in_specs=[pl.BlockSpec((d, 128), lambda c: (0, c))],
        out_specs=pl.BlockSpec((d, 128), lambda c: (c, 0)),
        out_shape=jax.ShapeDtypeStruct((vw, 128), jnp.float32),
    )


def _detile_fn(v, d, n_workers):
    ch = 128
    n_full = v // ch
    tail = v - n_full * ch
    assert tail % _L == 0 and tail > 0
    per_lo = n_full // n_workers
    n_extra = n_full - per_lo * n_workers
    nc = per_lo + 2
    assert nc % 2 == 0
    mesh = plsc.VectorSubcoreMesh(core_axis_name="c", subcore_axis_name="s")

    @functools.partial(
        pl.kernel,
        mesh=mesh,
        out_type=jax.ShapeDtypeStruct((v * d,), jnp.float32),
        compiler_params=pltpu.CompilerParams(needs_layout_passes=False),
        scratch_types=[
            *[pltpu.VMEM((d, ch), jnp.float32) for _ in range(2)],
            *[pltpu.VMEM((ch * d,), jnp.float32) for _ in range(2)],
            pltpu.VMEM((d, tail), jnp.float32),
            pltpu.VMEM((tail * d,), jnp.float32),
            *[pltpu.SemaphoreType.DMA for _ in range(4)],
        ],
    )
    def k(tt_hbm, out_hbm, tb0, tb1, ob0, ob1, ttb, otb, *sems):
        tbs, obs = (tb0, tb1), (ob0, ob1)
        isem = sems[:2]
        osem = sems[2:]
        wid = lax.axis_index("s") * 2 + lax.axis_index("c")
        base_c = wid * per_lo + jnp.minimum(wid, n_extra)
        iota = lax.broadcasted_iota(jnp.int32, (_L,), 0)
        iota_d = iota * d

        def blk(r):
            return jnp.minimum(base_c + r, n_full - 1)

        def start_in(r, b):
            c = blk(r)
            pltpu.async_copy(tt_hbm.at[:, pl.ds(c * ch, ch)], tbs[b], isem[b])

        def wait_in(r, b):
            c = blk(r)
            pltpu.make_async_copy(tt_hbm.at[:, pl.ds(c * ch, ch)], tbs[b],
                                  isem[b]).wait()

        def out_ref(r, b):
            return out_hbm.at[pl.ds(blk(r) * ch * d, ch * d)]

        def transpose(tb, ob, width):
            def grp(j, carry):
                a0 = j * (_L * d)
                for dd in range(d):
                    vals = tb[dd, pl.ds(j * _L, _L)]
                    plsc.store_scatter(ob, [a0 + iota_d + dd], vals)
                return carry

            lax.fori_loop(0, width // _L, grp, 0)

        for b in range(2):
            start_in(b, b)

        def body(i, carry):
            for b in range(2):
                r = i * 2 + b
                wait_in(r, b)
                transpose(tbs[b], obs[b], ch)
                pltpu.async_copy(obs[b], out_ref(r, b), osem[b])

                @pl.when(i * 2 + b + 2 < nc)
                def _():
                    pltpu.make_async_copy(obs[b], out_ref(r, b),
                                          osem[b]).wait()
                    start_in(r + 2, b)

            return carry

        lax.fori_loop(0, nc // 2, body, 0)
        for b in range(2):
            pltpu.make_async_copy(obs[b], out_ref(nc - 2 + b, b),
                                  osem[b]).wait()

        pltpu.sync_copy(tt_hbm.at[:, pl.ds(n_full * ch, tail)], ttb)
        transpose(ttb, otb, tail)
        pltpu.sync_copy(otb, out_hbm.at[pl.ds(n_full * ch * d, tail * d)])

    return k


def _gather_fn(n_batch, seq, d, n_workers):
    w = 2 * d
    rows_per_w = n_batch // n_workers
    idx_per_w = rows_per_w * seq
    n_chunks = rows_per_w
    assert n_chunks % _NBUF == 0 and n_chunks // _NBUF >= 2
    mesh = plsc.VectorSubcoreMesh(core_axis_name="c", subcore_axis_name="s")

    @functools.partial(
        pl.kernel,
        mesh=mesh,
        out_type=jax.ShapeDtypeStruct((n_batch, seq, w), jnp.float32),
        compiler_params=pltpu.CompilerParams(use_tc_tiling_on_sc=False,
                                             needs_layout_passes=False),
        scratch_types=[
            pltpu.VMEM((idx_per_w,), jnp.int32),
            pltpu.VMEM((idx_per_w,), jnp.int32),
            *[pltpu.VMEM((seq, d), jnp.float32) for _ in range(_NBUF)],
            *[pltpu.SemaphoreType.DMA for _ in range(2 * _NBUF)],
        ],
    )
    def k(idx_hbm, table_hbm, out_hbm, idx_v, q_v, *bufs_and_sems):
        rows_v = bufs_and_sems[:_NBUF]
        gsem = bufs_and_sems[_NBUF:2 * _NBUF]
        osem = bufs_and_sems[2 * _NBUF:]
        wid = lax.axis_index("s") * 2 + lax.axis_index("c")
        base = wid * idx_per_w
        row0 = wid * rows_per_w

        def idx_slice(g):
            return q_v.at[pl.ds(g * seq, seq)]

        def start_gather(g, b):
            pltpu.async_copy(table_hbm.at[idx_slice(g)], rows_v[b], gsem[b])

        def wait_gather(g, b):
            pltpu.make_async_copy(table_hbm.at[idx_slice(g)], rows_v[b],
                                  gsem[b]).wait()

        def out_slice(g):
            return out_hbm.at[row0 + g, :, pl.ds(0, d)]

        pltpu.sync_copy(idx_hbm.at[pl.ds(base, idx_per_w)], idx_v)

        def pre(j, carry):
            vv = idx_v[pl.ds(j * 16, 16)]
            q_v[pl.ds(j * 16, 16)] = (
                ((vv >> 7) << 7) + ((vv & 63) << 1) + ((vv >> 6) & 1))
            return carry

        lax.fori_loop(0, idx_per_w // 16, pre, 0)

        for b in range(_NBUF):
            start_gather(b, b)

        def body(i, carry):
            for b in range(_NBUF):
                g = i * _NBUF + b
                wait_gather(g, b)
                pltpu.async_copy(rows_v[b], out_slice(g), osem[b])

                @pl.when(i * _NBUF + b + _NBUF < n_chunks)
                def _():
                    pltpu.make_async_copy(rows_v[b], out_slice(g),
                                          osem[b]).wait()
                    start_gather(g + _NBUF, b)

            return carry

        lax.fori_loop(0, n_chunks // _NBUF, body, 0)

        for b in range(_NBUF):
            g = n_chunks - _NBUF + b
            pltpu.make_async_copy(rows_v[b], out_slice(g), osem[b]).wait()

    return k


def kernel(input_ids, table):
    b, s = input_ids.shape
    v, d = table.shape
    idx = input_ids.reshape(b * s).astype(jnp.int32)
    flat_table = _tc_detile_fn(v, d)(table.T)
    vp = flat_table.shape[0] * 2
    wide = _gather_fn(b, s, d, 32)(idx, flat_table.reshape(vp, d))
    return wide[:, :, :d]


# speedup vs baseline: 4.9008x; 4.9008x over previous
import functools

import jax
import jax.numpy as jnp
from jax import lax
from jax.experimental import pallas as pl
from jax.experimental.pallas import tpu as pltpu
from jax.experimental.pallas import tpu_sc as plsc

_NBUF = 4


def _gather_fn(n_batch, seq, d, n_workers):
    w = 2 * d
    rows_per_w = n_batch // n_workers
    idx_per_w = rows_per_w * seq
    n_chunks = rows_per_w
    assert n_chunks % _NBUF == 0 and n_chunks // _NBUF >= 2
    mesh = plsc.VectorSubcoreMesh(core_axis_name="c", subcore_axis_name="s")

    @functools.partial(
        pl.kernel,
        mesh=mesh,
        out_type=jax.ShapeDtypeStruct((n_batch, seq, w), jnp.float32),
        compiler_params=pltpu.CompilerParams(use_tc_tiling_on_sc=False),
        scratch_types=[
            pltpu.VMEM((idx_per_w,), jnp.int32),
            *[pltpu.VMEM((seq, d), jnp.float32) for _ in range(_NBUF)],
            *[pltpu.SemaphoreType.DMA for _ in range(2 * _NBUF)],
        ],
    )
    def k(idx_hbm, table_hbm, out_hbm, idx_v, *bufs_and_sems):
        rows_v = bufs_and_sems[:_NBUF]
        gsem = bufs_and_sems[_NBUF:2 * _NBUF]
        osem = bufs_and_sems[2 * _NBUF:]
        wid = lax.axis_index("s") * 2 + lax.axis_index("c")
        base = wid * idx_per_w
        row0 = wid * rows_per_w

        def idx_slice(g):
            return idx_v.at[pl.ds(g * seq, seq)]

        def start_gather(g, b):
            pltpu.async_copy(table_hbm.at[idx_slice(g)], rows_v[b], gsem[b])

        def wait_gather(g, b):
            pltpu.make_async_copy(table_hbm.at[idx_slice(g)], rows_v[b],
                                  gsem[b]).wait()

        def out_slice(g):
            return out_hbm.at[row0 + g, :, pl.ds(0, d)]

        pltpu.sync_copy(idx_hbm.at[pl.ds(base, idx_per_w)], idx_v)

        for b in range(_NBUF):
            start_gather(b, b)

        def body(i, carry):
            for b in range(_NBUF):
                g = i * _NBUF + b
                wait_gather(g, b)
                pltpu.async_copy(rows_v[b], out_slice(g), osem[b])

                @pl.when(i * _NBUF + b + _NBUF < n_chunks)
                def _():
                    pltpu.make_async_copy(rows_v[b], out_slice(g),
                                          osem[b]).wait()
                    start_gather(g + _NBUF, b)

            return carry

        lax.fori_loop(0, n_chunks // _NBUF, body, 0)

        for b in range(_NBUF):
            g = n_chunks - _NBUF + b
            pltpu.make_async_copy(rows_v[b], out_slice(g), osem[b]).wait()

    return k


def kernel(input_ids, table):
    b, s = input_ids.shape
    v, d = table.shape
    idx = input_ids.reshape(b * s).astype(jnp.int32)
    wide = _gather_fn(b, s, d, 32)(idx, table)
    return wide[:, :, :d]


# speedup vs baseline: 4.9069x; 1.0012x over previous
import functools

import jax
import jax.numpy as jnp
from jax import lax
from jax.experimental import pallas as pl
from jax.experimental.pallas import tpu as pltpu
from jax.experimental.pallas import tpu_sc as plsc

_NBUF = 8


def _gather_fn(n_batch, seq, d, n_workers):
    w = 2 * d
    rows_per_w = n_batch // n_workers
    idx_per_w = rows_per_w * seq
    n_chunks = rows_per_w
    assert n_chunks % _NBUF == 0 and n_chunks // _NBUF >= 2
    mesh = plsc.VectorSubcoreMesh(core_axis_name="c", subcore_axis_name="s")

    @functools.partial(
        pl.kernel,
        mesh=mesh,
        out_type=jax.ShapeDtypeStruct((n_batch, seq, w), jnp.float32),
        compiler_params=pltpu.CompilerParams(use_tc_tiling_on_sc=False),
        scratch_types=[
            pltpu.VMEM((idx_per_w,), jnp.int32),
            *[pltpu.VMEM((seq, d), jnp.float32) for _ in range(_NBUF)],
            *[pltpu.SemaphoreType.DMA for _ in range(2 * _NBUF)],
        ],
    )
    def k(idx_hbm, table_hbm, out_hbm, idx_v, *bufs_and_sems):
        rows_v = bufs_and_sems[:_NBUF]
        gsem = bufs_and_sems[_NBUF:2 * _NBUF]
        osem = bufs_and_sems[2 * _NBUF:]
        wid = lax.axis_index("s") * 2 + lax.axis_index("c")
        base = wid * idx_per_w
        row0 = wid * rows_per_w

        def idx_slice(g):
            return idx_v.at[pl.ds(g * seq, seq)]

        def start_gather(g, b):
            pltpu.async_copy(table_hbm.at[idx_slice(g)], rows_v[b], gsem[b])

        def wait_gather(g, b):
            pltpu.make_async_copy(table_hbm.at[idx_slice(g)], rows_v[b],
                                  gsem[b]).wait()

        def out_slice(g):
            return out_hbm.at[row0 + g, :, pl.ds(0, d)]

        pltpu.sync_copy(idx_hbm.at[pl.ds(base, idx_per_w)], idx_v)

        for b in range(_NBUF):
            start_gather(b, b)

        def body(i, carry):
            for b in range(_NBUF):
                g = i * _NBUF + b
                wait_gather(g, b)
                pltpu.async_copy(rows_v[b], out_slice(g), osem[b])

                @pl.when(i * _NBUF + b + _NBUF < n_chunks)
                def _():
                    pltpu.make_async_copy(rows_v[b], out_slice(g),
                                          osem[b]).wait()
                    start_gather(g + _NBUF, b)

            return carry

        lax.fori_loop(0, n_chunks // _NBUF, body, 0)

        for b in range(_NBUF):
            g = n_chunks - _NBUF + b
            pltpu.make_async_copy(rows_v[b], out_slice(g), osem[b]).wait()

    return k


def kernel(input_ids, table):
    b, s = input_ids.shape
    v, d = table.shape
    idx = input_ids.reshape(b * s).astype(jnp.int32)
    wide = _gather_fn(b, s, d, 32)(idx, table)
    return wide[:, :, :d]
